# baseline (speedup 1.0000x reference)
import functools

import jax
import jax.numpy as jnp
from jax import lax
from jax.experimental import pallas as pl
from jax.experimental.pallas import tpu as pltpu
from jax.experimental.pallas import tpu_sc as plsc

B = 425984
D = 64
NC = 2
NS = 16
NW = NC * NS
BPW = B // NW
CH = 128
NCHUNK = BPW // CH

_mesh = plsc.VectorSubcoreMesh(core_axis_name="c", subcore_axis_name="s")


@functools.partial(
    pl.kernel,
    out_type=jax.ShapeDtypeStruct((B, D), jnp.float32),
    mesh=_mesh,
    scratch_types=[
        pltpu.VMEM((NCHUNK, CH), jnp.int32),
        pltpu.VMEM((CH, D), jnp.float32),
        pltpu.SemaphoreType.DMA,
    ],
    compiler_params=pltpu.CompilerParams(use_tc_tiling_on_sc=False),
)
def _gather_kernel(idx_hbm, table_hbm, out_hbm, idx_v, rows_v, sem):
    wid = lax.axis_index("s") * NC + lax.axis_index("c")
    base = wid * BPW
    pltpu.sync_copy(idx_hbm.at[wid], idx_v)

    def body(j, carry):
        pltpu.async_copy(table_hbm.at[idx_v.at[j]], rows_v, sem).wait()
        pltpu.sync_copy(rows_v, out_hbm.at[pl.ds(base + j * CH, CH)])
        return carry

    lax.fori_loop(0, NCHUNK, body, 0)


def kernel(nodes, kg_features):
    idx = nodes.astype(jnp.int32).reshape(NW, NCHUNK, CH)
    return _gather_kernel(idx, kg_features)


# speedup vs baseline: 1.0709x; 1.0709x over previous
import functools

import jax
import jax.numpy as jnp
from jax import lax
from jax.experimental import pallas as pl
from jax.experimental.pallas import tpu as pltpu
from jax.experimental.pallas import tpu_sc as plsc

B = 425984
D = 64
NC = 2
NS = 16
NW = NC * NS
BPW = B // NW
CH = 128
NCHUNK = BPW // CH
K = 4
NG = NCHUNK // K
NP = NG // 2

_mesh = plsc.VectorSubcoreMesh(core_axis_name="c", subcore_axis_name="s")


@functools.partial(
    pl.kernel,
    out_type=jax.ShapeDtypeStruct((B, D), jnp.float32),
    mesh=_mesh,
    scratch_types=[
        pltpu.VMEM((NCHUNK, CH), jnp.int32),
        pltpu.VMEM((K, CH, D), jnp.float32),
        pltpu.VMEM((K, CH, D), jnp.float32),
        pltpu.SemaphoreType.DMA,
        pltpu.SemaphoreType.DMA,
        pltpu.SemaphoreType.DMA,
        pltpu.SemaphoreType.DMA,
    ],
    compiler_params=pltpu.CompilerParams(use_tc_tiling_on_sc=False),
)
def _gather_kernel(idx_hbm, table_hbm, out_hbm, idx_v, rows0, rows1,
                   gsem0, gsem1, wsem0, wsem1):
    wid = lax.axis_index("s") * NC + lax.axis_index("c")
    base = wid * BPW
    pltpu.sync_copy(idx_hbm.at[wid], idx_v)

    def fire_gathers(g, rows, gsem):
        for b in range(K):
            pltpu.async_copy(table_hbm.at[idx_v.at[g * K + b]], rows.at[b], gsem)

    def drain_gathers(g, rows, gsem):
        for b in range(K):
            pltpu.make_async_copy(
                table_hbm.at[idx_v.at[g * K + b]], rows.at[b], gsem).wait()

    def fire_writes(g, rows, wsem):
        for b in range(K):
            pltpu.async_copy(
                rows.at[b], out_hbm.at[pl.ds(base + (g * K + b) * CH, CH)], wsem)

    def drain_writes(g, rows, wsem):
        for b in range(K):
            pltpu.make_async_copy(
                rows.at[b], out_hbm.at[pl.ds(base + (g * K + b) * CH, CH)],
                wsem).wait()

    fire_gathers(0, rows0, gsem0)
    fire_gathers(1, rows1, gsem1)

    def body(p, carry):
        drain_gathers(2 * p, rows0, gsem0)
        fire_writes(2 * p, rows0, wsem0)
        drain_gathers(2 * p + 1, rows1, gsem1)
        fire_writes(2 * p + 1, rows1, wsem1)
        drain_writes(2 * p, rows0, wsem0)
        fire_gathers(2 * p + 2, rows0, gsem0)
        drain_writes(2 * p + 1, rows1, wsem1)
        fire_gathers(2 * p + 3, rows1, gsem1)
        return carry

    lax.fori_loop(0, NP - 1, body, 0)

    p = NP - 1
    drain_gathers(2 * p, rows0, gsem0)
    fire_writes(2 * p, rows0, wsem0)
    drain_gathers(2 * p + 1, rows1, gsem1)
    fire_writes(2 * p + 1, rows1, wsem1)
    drain_writes(2 * p, rows0, wsem0)
    drain_writes(2 * p + 1, rows1, wsem1)


def kernel(nodes, kg_features):
    idx = nodes.astype(jnp.int32).reshape(NW, NCHUNK, CH)
    return _gather_kernel(idx, kg_features)
